# Initial kernel scaffold; baseline (speedup 1.0000x reference)
import jax
import jax.numpy as jnp
from jax.experimental import pallas as pl


def kernel(x, edge_index, edge_type, W_edge, W_msg, b_msg, W_upd, b_upd, gamma, beta):
    raise NotImplementedError("write your pallas kernel here")


# speedup vs baseline: 3.4576x; 3.4576x over previous
import functools

import jax
import jax.numpy as jnp
from jax import lax
from jax.experimental import pallas as pl
from jax.experimental.pallas import tpu as pltpu
from jax.experimental.pallas import tpu_sc as plsc

N_NODES = 10000
HIDDEN = 256
NUM_TYPES = 16
N_EDGES = 160000
HALF = HIDDEN // 2

NC = 2
NS = 16
EPW = N_EDGES // NS
K = 80
NBLK = EPW // K
N_PAD = 10240
RPS = N_PAD // NS
DEG_HALF = N_PAD // NC
DEG_ROWS = 5248
DRPS = DEG_ROWS // NS

_PREC = lax.Precision.HIGHEST


def _stage1_body(x_ref, wx_ref, we_ref, wedge_ref, bmsg_ref, comb_ref):
    xb = x_ref[...]
    y = lax.dot_general(xb, wx_ref[...], (((1,), (1,)), ((), ())),
                        precision=_PREC, preferred_element_type=jnp.float32)
    t = lax.dot_general(wedge_ref[...], we_ref[...], (((1,), (1,)), ((), ())),
                        precision=_PREC, preferred_element_type=jnp.float32)
    t = t + bmsg_ref[...]
    nb = xb.shape[0]
    m = jax.nn.relu(y[:, None, :] + t[None, :, :])
    m2 = m.reshape(nb * NUM_TYPES, HIDDEN)
    comb_ref[0, :, :] = m2[:, :HALF]
    comb_ref[1, :, :] = m2[:, HALF:]


def _stage1(x, wx, we, wedge, bmsg):
    nb = 200
    grid = N_NODES // nb
    return pl.pallas_call(
        _stage1_body,
        grid=(grid,),
        in_specs=[
            pl.BlockSpec((nb, HIDDEN), lambda i: (i, 0)),
            pl.BlockSpec((HIDDEN, HIDDEN), lambda i: (0, 0)),
            pl.BlockSpec((HIDDEN, NUM_TYPES), lambda i: (0, 0)),
            pl.BlockSpec((NUM_TYPES, NUM_TYPES), lambda i: (0, 0)),
            pl.BlockSpec((1, HIDDEN), lambda i: (0, 0)),
        ],
        out_specs=pl.BlockSpec((NC, nb * NUM_TYPES, HALF), lambda i: (0, i, 0)),
        out_shape=jax.ShapeDtypeStruct((NC, N_NODES * NUM_TYPES, HALF),
                                       jnp.float32),
    )(x, wx, we, wedge, bmsg)


def _sc_body(comb_hbm, cidx_hbm, dst_hbm, agg_out, deg_out,
             cidx_v, dst_v, rows_v, ones_v, agg_sh):
    c = lax.axis_index("c")
    s = lax.axis_index("s")
    zero16 = jnp.zeros((16,), jnp.float32)
    my_rows = pl.ds(s * RPS, RPS)

    for i in range(8):
        for j in range(0, HALF, 16):
            rows_v[i, pl.ds(j, 16)] = zero16

    @pl.loop(0, RPS, step=8)
    def _(r):
        pltpu.sync_copy(rows_v.at[pl.ds(0, 8)], agg_sh.at[pl.ds(s * RPS + r, 8)])

    for i in range(K):
        for j in range(0, HALF, 16):
            ones_v[i, pl.ds(j, 16)] = zero16 + 1.0

    plsc.subcore_barrier()

    base = s * EPW
    cbase = c * (N_NODES * NUM_TYPES)

    @pl.loop(0, NBLK)
    def _(b):
        off = base + b * K
        pltpu.sync_copy(cidx_hbm.at[pl.ds(off, K)], cidx_v)
        pltpu.sync_copy(dst_hbm.at[pl.ds(off, K)], dst_v)

        @pl.loop(0, K, step=16)
        def _(j):
            cidx_v[pl.ds(j, 16)] = cidx_v[pl.ds(j, 16)] + cbase

        pltpu.sync_copy(comb_hbm.at[cidx_v], rows_v)
        pltpu.sync_copy(rows_v, agg_sh.at[dst_v], add=True)

    plsc.subcore_barrier()
    pltpu.sync_copy(agg_sh.at[my_rows], agg_out.at[pl.ds(c * N_PAD + s * RPS, RPS)])

    for i in range(8):
        for j in range(0, HALF, 16):
            rows_v[i, pl.ds(j, 16)] = zero16

    @pl.loop(0, RPS, step=8)
    def _(r):
        pltpu.sync_copy(rows_v.at[pl.ds(0, 8)], agg_sh.at[pl.ds(s * RPS + r, 8)])

    plsc.subcore_barrier()

    @pl.loop(0, NBLK)
    def _(b):
        off = base + b * K
        pltpu.sync_copy(dst_hbm.at[pl.ds(off, K)], dst_v)
        pltpu.sync_copy(ones_v, agg_sh.at[dst_v], add=True)

    plsc.subcore_barrier()

    @pl.when(c == 0)
    def _():
        pltpu.sync_copy(agg_sh.at[my_rows], deg_out.at[my_rows])


def _sc_aggregate(comb, cidx, dst):
    mesh = plsc.VectorSubcoreMesh(core_axis_name="c", subcore_axis_name="s")
    run = pl.kernel(
        _sc_body,
        out_type=(jax.ShapeDtypeStruct((NC * N_PAD, HALF), jnp.float32),
                  jax.ShapeDtypeStruct((N_PAD, HALF), jnp.float32)),
        mesh=mesh,
        scratch_types=[
            pltpu.VMEM((K,), jnp.int32),
            pltpu.VMEM((K,), jnp.int32),
            pltpu.VMEM((K, HALF), jnp.float32),
            pltpu.VMEM((K, HALF), jnp.float32),
            pltpu.VMEM_SHARED((N_PAD, HALF), jnp.float32),
        ],
    )
    comb2 = comb.reshape(NC * N_NODES * NUM_TYPES, HALF)
    agg, deg = run(comb2, cidx, dst)
    return agg.reshape(NC, N_PAD, HALF), deg[:N_NODES]


def _stage3_body(x_ref, agg_ref, deg_ref, wupd_ref, bupd_ref, g_ref, b_ref,
                 out_ref):
    deg = jnp.maximum(deg_ref[:, 0:1], 1.0)
    a = jnp.concatenate([agg_ref[0], agg_ref[1]], axis=-1) / deg
    upd = lax.dot_general(a, wupd_ref[...], (((1,), (1,)), ((), ())),
                          precision=_PREC, preferred_element_type=jnp.float32)
    h = x_ref[...] + upd + bupd_ref[...]
    mean = jnp.mean(h, axis=-1, keepdims=True)
    var = jnp.mean((h - mean) ** 2, axis=-1, keepdims=True)
    out_ref[...] = (h - mean) * lax.rsqrt(var + 1e-5) * g_ref[...] + b_ref[...]


def _stage3(x, agg2, deg, wupd, bupd, gamma, beta):
    nb = 400
    grid = N_NODES // nb
    return pl.pallas_call(
        _stage3_body,
        grid=(grid,),
        in_specs=[
            pl.BlockSpec((nb, HIDDEN), lambda i: (i, 0)),
            pl.BlockSpec((NC, nb, HALF), lambda i: (0, i, 0)),
            pl.BlockSpec((nb, HALF), lambda i: (i, 0)),
            pl.BlockSpec((HIDDEN, HIDDEN), lambda i: (0, 0)),
            pl.BlockSpec((1, HIDDEN), lambda i: (0, 0)),
            pl.BlockSpec((1, HIDDEN), lambda i: (0, 0)),
            pl.BlockSpec((1, HIDDEN), lambda i: (0, 0)),
        ],
        out_specs=pl.BlockSpec((nb, HIDDEN), lambda i: (i, 0)),
        out_shape=jax.ShapeDtypeStruct((N_NODES, HIDDEN), jnp.float32),
    )(x, agg2, deg, wupd, bupd, gamma, beta)


def kernel(x, edge_index, edge_type, W_edge, W_msg, b_msg, W_upd, b_upd,
           gamma, beta):
    src = edge_index[0].astype(jnp.int32)
    dst = edge_index[1].astype(jnp.int32)
    et = edge_type.astype(jnp.int32)
    cidx = src * NUM_TYPES + et

    comb = _stage1(x, W_msg[:, :HIDDEN], W_msg[:, HIDDEN:], W_edge,
                   b_msg.reshape(1, HIDDEN))
    agg2, deg = _sc_aggregate(comb, cidx, dst)
    return _stage3(x, agg2, deg, W_upd, b_upd.reshape(1, HIDDEN),
                   gamma.reshape(1, HIDDEN), beta.reshape(1, HIDDEN))


# speedup vs baseline: 5.1069x; 1.4770x over previous
import functools

import jax
import jax.numpy as jnp
from jax import lax
from jax.experimental import pallas as pl
from jax.experimental.pallas import tpu as pltpu
from jax.experimental.pallas import tpu_sc as plsc

N_NODES = 10000
HIDDEN = 256
NUM_TYPES = 16
N_EDGES = 160000
HALF = HIDDEN // 2

NC = 2
NS = 16
EPW = N_EDGES // NS
K = 80
NBLK = EPW // K
N_PAD = 10240
RPS = N_PAD // NS
DEG_HALF = N_PAD // NC
DEG_ROWS = 5248
DRPS = DEG_ROWS // NS

_PREC = lax.Precision.HIGHEST


def _stage1_body(x_ref, wx_ref, we_ref, wedge_ref, bmsg_ref, comb_ref):
    xb = x_ref[...]
    y = lax.dot_general(xb, wx_ref[...], (((1,), (1,)), ((), ())),
                        precision=_PREC, preferred_element_type=jnp.float32)
    t = lax.dot_general(wedge_ref[...], we_ref[...], (((1,), (1,)), ((), ())),
                        precision=_PREC, preferred_element_type=jnp.float32)
    t = t + bmsg_ref[...]
    nb = xb.shape[0]
    m = jax.nn.relu(y[:, None, :] + t[None, :, :])
    m2 = m.reshape(nb * NUM_TYPES, HIDDEN)
    comb_ref[0, :, :] = m2[:, :HALF]
    comb_ref[1, :, :] = m2[:, HALF:]


def _stage1(x, wx, we, wedge, bmsg):
    nb = 200
    grid = N_NODES // nb
    return pl.pallas_call(
        _stage1_body,
        grid=(grid,),
        in_specs=[
            pl.BlockSpec((nb, HIDDEN), lambda i: (i, 0)),
            pl.BlockSpec((HIDDEN, HIDDEN), lambda i: (0, 0)),
            pl.BlockSpec((HIDDEN, NUM_TYPES), lambda i: (0, 0)),
            pl.BlockSpec((NUM_TYPES, NUM_TYPES), lambda i: (0, 0)),
            pl.BlockSpec((1, HIDDEN), lambda i: (0, 0)),
        ],
        out_specs=pl.BlockSpec((NC, nb * NUM_TYPES, HALF), lambda i: (0, i, 0)),
        out_shape=jax.ShapeDtypeStruct((NC, N_NODES * NUM_TYPES, HALF),
                                       jnp.float32),
    )(x, wx, we, wedge, bmsg)


def _sc_body(comb_hbm, cidx_hbm, dst_hbm, agg_out, deg_out,
             cidx_v0, cidx_v1, dst_v0, dst_v1,
             rows0, rows1, ones_v, agg_sh,
             sem_z, sem_g0, sem_g1, sem_s0, sem_s1):
    c = lax.axis_index("c")
    s = lax.axis_index("s")
    zero16 = jnp.zeros((16,), jnp.float32)
    my_rows = pl.ds(s * RPS, RPS)
    cidx_v = (cidx_v0, cidx_v1)
    dst_v = (dst_v0, dst_v1)
    rows = (rows0, rows1)
    sem_g = (sem_g0, sem_g1)
    sem_s = (sem_s0, sem_s1)

    def zero_my_slice():
        for i in range(40):
            for j in range(0, HALF, 16):
                rows0[i, pl.ds(j, 16)] = zero16
        cps = [pltpu.async_copy(rows0.at[pl.ds(0, 40)],
                                agg_sh.at[pl.ds(s * RPS + r, 40)], sem_z)
               for r in range(0, RPS, 40)]
        for cp in cps:
            cp.wait()

    zero_my_slice()

    for i in range(K):
        for j in range(0, HALF, 16):
            ones_v[i, pl.ds(j, 16)] = zero16 + 1.0

    cbase = (c * NS + s) * EPW
    dbase = s * EPW

    plsc.subcore_barrier()

    for t in range(2):
        pltpu.sync_copy(cidx_hbm.at[pl.ds(cbase + t * K, K)], cidx_v[t])
        pltpu.sync_copy(dst_hbm.at[pl.ds(dbase + t * K, K)], dst_v[t])
        pltpu.async_copy(comb_hbm.at[cidx_v[t]], rows[t], sem_g[t])

    @pl.loop(0, NBLK - 3, step=2)
    def _(b):
        for t in range(2):
            pltpu.make_async_copy(comb_hbm.at[cidx_v[t]], rows[t],
                                  sem_g[t]).wait()
            pltpu.async_copy(rows[t], agg_sh.at[dst_v[t]], sem_s[t], add=True)
            pltpu.make_async_copy(rows[t], agg_sh.at[dst_v[t]],
                                  sem_s[t]).wait()
            off2 = (b + t + 2) * K
            pltpu.sync_copy(cidx_hbm.at[pl.ds(cbase + off2, K)], cidx_v[t])
            pltpu.sync_copy(dst_hbm.at[pl.ds(dbase + off2, K)], dst_v[t])
            pltpu.async_copy(comb_hbm.at[cidx_v[t]], rows[t], sem_g[t])

    for t in range(2):
        pltpu.make_async_copy(comb_hbm.at[cidx_v[t]], rows[t], sem_g[t]).wait()
        pltpu.async_copy(rows[t], agg_sh.at[dst_v[t]], sem_s[t], add=True)
        pltpu.make_async_copy(rows[t], agg_sh.at[dst_v[t]], sem_s[t]).wait()

    pltpu.sync_copy(cidx_hbm.at[pl.ds(cbase + (NBLK - 1) * K, K)], cidx_v0)
    pltpu.sync_copy(dst_hbm.at[pl.ds(dbase + (NBLK - 1) * K, K)], dst_v0)
    pltpu.sync_copy(comb_hbm.at[cidx_v0], rows0)
    pltpu.sync_copy(rows0, agg_sh.at[dst_v0], add=True)

    plsc.subcore_barrier()
    pltpu.sync_copy(agg_sh.at[my_rows], agg_out.at[pl.ds(c * N_PAD + s * RPS, RPS)])

    zero_my_slice()
    plsc.subcore_barrier()

    for t in range(2):
        pltpu.sync_copy(dst_hbm.at[pl.ds(dbase + t * K, K)], dst_v[t])
        pltpu.async_copy(ones_v, agg_sh.at[dst_v[t]], sem_g[t], add=True)

    @pl.loop(0, NBLK - 3, step=2)
    def _(b):
        for t in range(2):
            pltpu.make_async_copy(ones_v, agg_sh.at[dst_v[t]], sem_g[t]).wait()
            pltpu.sync_copy(dst_hbm.at[pl.ds(dbase + (b + t + 2) * K, K)],
                            dst_v[t])
            pltpu.async_copy(ones_v, agg_sh.at[dst_v[t]], sem_g[t], add=True)

    for t in range(2):
        pltpu.make_async_copy(ones_v, agg_sh.at[dst_v[t]], sem_g[t]).wait()

    pltpu.sync_copy(dst_hbm.at[pl.ds(dbase + (NBLK - 1) * K, K)], dst_v0)
    pltpu.sync_copy(ones_v, agg_sh.at[dst_v0], add=True)

    plsc.subcore_barrier()

    @pl.when(c == 0)
    def _():
        pltpu.sync_copy(agg_sh.at[my_rows], deg_out.at[my_rows])


def _sc_aggregate(comb, cidx, dst):
    mesh = plsc.VectorSubcoreMesh(core_axis_name="c", subcore_axis_name="s")
    run = pl.kernel(
        _sc_body,
        out_type=(jax.ShapeDtypeStruct((NC * N_PAD, HALF), jnp.float32),
                  jax.ShapeDtypeStruct((N_PAD, HALF), jnp.float32)),
        mesh=mesh,
        scratch_types=[
            pltpu.VMEM((K,), jnp.int32),
            pltpu.VMEM((K,), jnp.int32),
            pltpu.VMEM((K,), jnp.int32),
            pltpu.VMEM((K,), jnp.int32),
            pltpu.VMEM((K, HALF), jnp.float32),
            pltpu.VMEM((K, HALF), jnp.float32),
            pltpu.VMEM((K, HALF), jnp.float32),
            pltpu.VMEM_SHARED((N_PAD, HALF), jnp.float32),
            pltpu.SemaphoreType.DMA,
            pltpu.SemaphoreType.DMA,
            pltpu.SemaphoreType.DMA,
            pltpu.SemaphoreType.DMA,
            pltpu.SemaphoreType.DMA,
        ],
    )
    comb2 = comb.reshape(NC * N_NODES * NUM_TYPES, HALF)
    cidx2 = jnp.concatenate([cidx, cidx + N_NODES * NUM_TYPES], axis=0)
    agg, deg = run(comb2, cidx2, dst)
    return agg.reshape(NC, N_PAD, HALF), deg[:N_NODES]


def _stage3_body(x_ref, agg_ref, deg_ref, wupd_ref, bupd_ref, g_ref, b_ref,
                 out_ref):
    deg = jnp.maximum(deg_ref[:, 0:1], 1.0)
    a = jnp.concatenate([agg_ref[0], agg_ref[1]], axis=-1) / deg
    upd = lax.dot_general(a, wupd_ref[...], (((1,), (1,)), ((), ())),
                          precision=_PREC, preferred_element_type=jnp.float32)
    h = x_ref[...] + upd + bupd_ref[...]
    mean = jnp.mean(h, axis=-1, keepdims=True)
    var = jnp.mean((h - mean) ** 2, axis=-1, keepdims=True)
    out_ref[...] = (h - mean) * lax.rsqrt(var + 1e-5) * g_ref[...] + b_ref[...]


def _stage3(x, agg2, deg, wupd, bupd, gamma, beta):
    nb = 400
    grid = N_NODES // nb
    return pl.pallas_call(
        _stage3_body,
        grid=(grid,),
        in_specs=[
            pl.BlockSpec((nb, HIDDEN), lambda i: (i, 0)),
            pl.BlockSpec((NC, nb, HALF), lambda i: (0, i, 0)),
            pl.BlockSpec((nb, HALF), lambda i: (i, 0)),
            pl.BlockSpec((HIDDEN, HIDDEN), lambda i: (0, 0)),
            pl.BlockSpec((1, HIDDEN), lambda i: (0, 0)),
            pl.BlockSpec((1, HIDDEN), lambda i: (0, 0)),
            pl.BlockSpec((1, HIDDEN), lambda i: (0, 0)),
        ],
        out_specs=pl.BlockSpec((nb, HIDDEN), lambda i: (i, 0)),
        out_shape=jax.ShapeDtypeStruct((N_NODES, HIDDEN), jnp.float32),
    )(x, agg2, deg, wupd, bupd, gamma, beta)


def kernel(x, edge_index, edge_type, W_edge, W_msg, b_msg, W_upd, b_upd,
           gamma, beta):
    src = edge_index[0].astype(jnp.int32)
    dst = edge_index[1].astype(jnp.int32)
    et = edge_type.astype(jnp.int32)
    cidx = src * NUM_TYPES + et

    comb = _stage1(x, W_msg[:, :HIDDEN], W_msg[:, HIDDEN:], W_edge,
                   b_msg.reshape(1, HIDDEN))
    agg2, deg = _sc_aggregate(comb, cidx, dst)
    return _stage3(x, agg2, deg, W_upd, b_upd.reshape(1, HIDDEN),
                   gamma.reshape(1, HIDDEN), beta.reshape(1, HIDDEN))
